# baseline (speedup 1.0000x reference)
import functools

import jax
import jax.numpy as jnp
from jax import lax
from jax.experimental import pallas as pl
from jax.experimental.pallas import tpu as pltpu
from jax.experimental.pallas import tpu_sc as plsc

N = 10000
E = 320000
D_IN = 128
D_H = 64
N_CLS = 7

NC = 2
NS = 16
NW = NC * NS
N_PAD = 10240
EPT = E // NW
CH = 4000
NCHUNK = E // CH
UNROLL = 10
BLKC = 1024


def _sc_params():
    return dict(
        mesh=plsc.VectorSubcoreMesh(core_axis_name="c", subcore_axis_name="s"),
        compiler_params=pltpu.CompilerParams(needs_layout_passes=False),
    )


def _deg_body(pk1d, out, dstv, hist):
    cid = lax.axis_index("c")
    sid = lax.axis_index("s")
    wid = cid * NS + sid
    pltpu.sync_copy(pk1d.at[pl.ds(wid * EPT, EPT)], dstv)

    def zero(i, c):
        hist[pl.ds(i * 16, 16)] = jnp.zeros((16,), jnp.float32)
        return c

    lax.fori_loop(0, N_PAD // 16, zero, 0)
    ones = jnp.ones((16,), jnp.float32)

    def body(k, c):
        d16 = lax.shift_right_logical(dstv[pl.ds(k * 16, 16)], 16)
        plsc.addupdate_scatter(hist, [d16], ones)
        return c

    lax.fori_loop(0, EPT // 16, body, 0)
    pltpu.sync_copy(hist, out.at[pl.ds(wid * N_PAD, N_PAD)])


@functools.cache
def _deg_call():
    return pl.kernel(
        _deg_body,
        out_type=jax.ShapeDtypeStruct((NW * N_PAD,), jnp.float32),
        scratch_types=[
            pltpu.VMEM((EPT,), jnp.int32),
            pltpu.VMEM((N_PAD,), jnp.float32),
        ],
        **_sc_params(),
    )


def _agg_body(ysT, pk1d, out,
              pkA, pkB, ys0, ys1, acc0, acc1, semA, semB):
    cid = lax.axis_index("c")
    sid = lax.axis_index("s")
    wid = cid * NS + sid
    f0 = 2 * wid * N_PAD
    off = lax.div(wid * NCHUNK, NW)

    def _start(c, pv, sem):
        cc = lax.rem(c + off, NCHUNK)
        pltpu.async_copy(pk1d.at[pl.ds(cc * CH, CH)], pv, sem)

    def _wait(pv, sem):
        pltpu.make_async_copy(pk1d.at[pl.ds(0, CH)], pv, sem).wait()

    def _process(pv):
        def body(i, c2):
            base = i * (UNROLL * 16)
            for u in range(UNROLL):
                w16 = pv[pl.ds(base + u * 16, 16)]
                s16 = jnp.bitwise_and(w16, 0xFFFF)
                d16 = lax.shift_right_logical(w16, 16)
                v0 = plsc.load_gather(ys0, [s16])
                v1 = plsc.load_gather(ys1, [s16])
                plsc.addupdate_scatter(acc0, [d16], v0)
                plsc.addupdate_scatter(acc1, [d16], v1)
            return c2

        lax.fori_loop(0, CH // 16 // UNROLL, body, 0)

    pltpu.sync_copy(ysT.at[pl.ds(f0, N_PAD)], ys0)
    pltpu.sync_copy(ysT.at[pl.ds(f0 + N_PAD, N_PAD)], ys1)
    pltpu.sync_copy(ysT.at[pl.ds(f0, N_PAD)], acc0)
    pltpu.sync_copy(ysT.at[pl.ds(f0 + N_PAD, N_PAD)], acc1)
    _start(0, pkA, semA)

    def chunk_pair(p, carry):
        c0 = 2 * p
        _start(c0 + 1, pkB, semB)
        _wait(pkA, semA)
        _process(pkA)
        _start(lax.rem(c0 + 2, NCHUNK), pkA, semA)
        _wait(pkB, semB)
        _process(pkB)
        return carry

    lax.fori_loop(0, NCHUNK // 2, chunk_pair, 0)
    _wait(pkA, semA)
    pltpu.sync_copy(acc0, out.at[pl.ds(f0, N_PAD)])
    pltpu.sync_copy(acc1, out.at[pl.ds(f0 + N_PAD, N_PAD)])


@functools.cache
def _agg_call():
    return pl.kernel(
        _agg_body,
        out_type=jax.ShapeDtypeStruct((D_H * N_PAD,), jnp.float32),
        scratch_types=[
            pltpu.VMEM((CH,), jnp.int32),
            pltpu.VMEM((CH,), jnp.int32),
            pltpu.VMEM((N_PAD,), jnp.float32),
            pltpu.VMEM((N_PAD,), jnp.float32),
            pltpu.VMEM((N_PAD,), jnp.float32),
            pltpu.VMEM((N_PAD,), jnp.float32),
            pltpu.SemaphoreType.DMA,
            pltpu.SemaphoreType.DMA,
        ],
        **_sc_params(),
    )


def _prescale_kernel(xT_ref, w1t_ref, deg_ref, ys_ref, dinv_ref):
    i = pl.program_id(0)
    dsum = jnp.sum(deg_ref[...], axis=0, keepdims=True) + 1.0
    cols = lax.broadcasted_iota(jnp.int32, (1, BLKC), 1) + i * BLKC
    dinv = jnp.where(cols < N, lax.rsqrt(dsum), 0.0)
    dinv_ref[...] = dinv
    y = jnp.dot(w1t_ref[...], xT_ref[...], preferred_element_type=jnp.float32)
    ys_ref[...] = y * dinv


def _prescale(xT_pad, w1t, deg):
    return pl.pallas_call(
        _prescale_kernel,
        grid=(N_PAD // BLKC,),
        in_specs=[
            pl.BlockSpec((D_IN, BLKC), lambda i: (0, i)),
            pl.BlockSpec((D_H, D_IN), lambda i: (0, 0)),
            pl.BlockSpec((NW, BLKC), lambda i: (0, i)),
        ],
        out_specs=[
            pl.BlockSpec((D_H, BLKC), lambda i: (0, i)),
            pl.BlockSpec((1, BLKC), lambda i: (0, i)),
        ],
        out_shape=[
            jax.ShapeDtypeStruct((D_H, N_PAD), jnp.float32),
            jax.ShapeDtypeStruct((1, N_PAD), jnp.float32),
        ],
    )(xT_pad, w1t, deg)


def _mid_kernel(agg_ref, dinv_ref, b1_ref, w2t_ref, ys2_ref):
    dinv = dinv_ref[...]
    x1 = jnp.maximum(agg_ref[...] * dinv + b1_ref[...], 0.0)
    y2 = jnp.dot(w2t_ref[...], x1, preferred_element_type=jnp.float32)
    ys2_ref[...] = y2 * dinv


def _mid(aggT, dinv, b1col, w2t):
    return pl.pallas_call(
        _mid_kernel,
        grid=(N_PAD // BLKC,),
        in_specs=[
            pl.BlockSpec((D_H, BLKC), lambda i: (0, i)),
            pl.BlockSpec((1, BLKC), lambda i: (0, i)),
            pl.BlockSpec((D_H, 1), lambda i: (0, 0)),
            pl.BlockSpec((D_H, D_H), lambda i: (0, 0)),
        ],
        out_specs=pl.BlockSpec((D_H, BLKC), lambda i: (0, i)),
        out_shape=jax.ShapeDtypeStruct((D_H, N_PAD), jnp.float32),
    )(aggT, dinv, b1col, w2t)


def _head_kernel(agg_ref, dinv_ref, b2_ref, wpt_ref, bp_ref, out_ref):
    x2 = jnp.maximum(agg_ref[...] * dinv_ref[...] + b2_ref[...], 0.0)
    logits = jnp.dot(wpt_ref[...], x2, preferred_element_type=jnp.float32)
    logits = logits + bp_ref[...]
    rows = lax.broadcasted_iota(jnp.int32, (8, BLKC), 0)
    valid = rows < N_CLS
    m = jnp.max(jnp.where(valid, logits, -1e30), axis=0, keepdims=True)
    s = jnp.sum(jnp.where(valid, jnp.exp(logits - m), 0.0), axis=0, keepdims=True)
    out_ref[...] = logits - m - jnp.log(s)


def _head(aggT, dinv, b2col, wpt_pad, bp_col):
    return pl.pallas_call(
        _head_kernel,
        grid=(N_PAD // BLKC,),
        in_specs=[
            pl.BlockSpec((D_H, BLKC), lambda i: (0, i)),
            pl.BlockSpec((1, BLKC), lambda i: (0, i)),
            pl.BlockSpec((D_H, 1), lambda i: (0, 0)),
            pl.BlockSpec((8, D_H), lambda i: (0, 0)),
            pl.BlockSpec((8, 1), lambda i: (0, 0)),
        ],
        out_specs=pl.BlockSpec((8, BLKC), lambda i: (0, i)),
        out_shape=jax.ShapeDtypeStruct((8, N_PAD), jnp.float32),
    )(aggT, dinv, b2col, wpt_pad, bp_col)


def kernel(x, edge_index, W1, b1, W2, b2, Wp, bp):
    xT_pad = jnp.pad(x.T, ((0, 0), (0, N_PAD - N)))
    w1t = W1.T
    w2t = W2.T
    wpt_pad = jnp.pad(Wp.T, ((0, 8 - N_CLS), (0, 0)))
    bp_col = jnp.pad(bp, (0, 8 - N_CLS)).reshape(8, 1)
    b1col = b1.reshape(D_H, 1)
    b2col = b2.reshape(D_H, 1)

    pk1d = jnp.bitwise_or(edge_index[0],
                          jnp.left_shift(edge_index[1], 16))

    deg = _deg_call()(pk1d).reshape(NW, N_PAD)
    ys1T, dinv = _prescale(xT_pad, w1t, deg)
    agg1T = _agg_call()(ys1T.reshape(-1), pk1d).reshape(D_H, N_PAD)
    ys2T = _mid(agg1T, dinv, b1col, w2t)
    agg2T = _agg_call()(ys2T.reshape(-1), pk1d).reshape(D_H, N_PAD)
    outT = _head(agg2T, dinv, b2col, wpt_pad, bp_col)
    return outT[:N_CLS, :N].T


# speedup vs baseline: 2.0313x; 2.0313x over previous
import functools

import jax
import jax.numpy as jnp
from jax import lax
from jax.experimental import pallas as pl
from jax.experimental.pallas import tpu as pltpu
from jax.experimental.pallas import tpu_sc as plsc

N = 10000
E = 320000
D_IN = 128
D_H = 64
N_CLS = 7

NC = 2
NS = 16
NW = NC * NS
N_PAD = 10240
EPT = E // NW
CH = 4000
NCHUNK = E // CH
UNROLL = 10
BLKC = 1024


def _sc_params():
    return dict(
        mesh=plsc.VectorSubcoreMesh(core_axis_name="c", subcore_axis_name="s"),
        compiler_params=pltpu.CompilerParams(needs_layout_passes=False),
    )


def _deg_body(pk1d, out, dstv, hist):
    cid = lax.axis_index("c")
    sid = lax.axis_index("s")
    wid = cid * NS + sid
    pltpu.sync_copy(pk1d.at[pl.ds(wid * EPT, EPT)], dstv)

    def zero(i, c):
        hist[pl.ds(i * 16, 16)] = jnp.zeros((16,), jnp.float32)
        return c

    lax.fori_loop(0, N_PAD // 16, zero, 0)
    ones = jnp.ones((16,), jnp.float32)

    def body(k, c):
        d16 = lax.shift_right_logical(dstv[pl.ds(k * 16, 16)], 16)
        plsc.addupdate_scatter(hist, [d16], ones)
        return c

    lax.fori_loop(0, EPT // 16, body, 0)
    pltpu.sync_copy(hist, out.at[pl.ds(wid * N_PAD, N_PAD)])


@functools.cache
def _deg_call():
    return pl.kernel(
        _deg_body,
        out_type=jax.ShapeDtypeStruct((NW * N_PAD,), jnp.float32),
        scratch_types=[
            pltpu.VMEM((EPT,), jnp.int32),
            pltpu.VMEM((N_PAD,), jnp.float32),
        ],
        **_sc_params(),
    )


def _agg_body(ysT, pk1d, out,
              pkA, pkB, ys0, ys1, acc0, acc1, semA, semB):
    cid = lax.axis_index("c")
    sid = lax.axis_index("s")
    wid = cid * NS + sid
    f0 = 2 * wid * N_PAD
    off = lax.div(wid * NCHUNK, NW)

    def _start(c, pv, sem):
        cc = lax.rem(c + off, NCHUNK)
        pltpu.async_copy(pk1d.at[pl.ds(cc * CH, CH)], pv, sem)

    def _wait(pv, sem):
        pltpu.make_async_copy(pk1d.at[pl.ds(0, CH)], pv, sem).wait()

    def _process(pv):
        @plsc.parallel_loop(0, CH, step=16, unroll=UNROLL)
        def body(i):
            w16 = pv[pl.ds(i, 16)]
            s16 = jnp.bitwise_and(w16, 0xFFFF)
            d16 = lax.shift_right_logical(w16, 16)
            v0 = plsc.load_gather(ys0, [s16])
            v1 = plsc.load_gather(ys1, [s16])
            plsc.addupdate_scatter(acc0, [d16], v0)
            plsc.addupdate_scatter(acc1, [d16], v1)

    pltpu.sync_copy(ysT.at[pl.ds(f0, N_PAD)], ys0)
    pltpu.sync_copy(ysT.at[pl.ds(f0 + N_PAD, N_PAD)], ys1)
    pltpu.sync_copy(ysT.at[pl.ds(f0, N_PAD)], acc0)
    pltpu.sync_copy(ysT.at[pl.ds(f0 + N_PAD, N_PAD)], acc1)
    _start(0, pkA, semA)

    def chunk_pair(p, carry):
        c0 = 2 * p
        _start(c0 + 1, pkB, semB)
        _wait(pkA, semA)
        _process(pkA)
        _start(lax.rem(c0 + 2, NCHUNK), pkA, semA)
        _wait(pkB, semB)
        _process(pkB)
        return carry

    lax.fori_loop(0, NCHUNK // 2, chunk_pair, 0)
    _wait(pkA, semA)
    pltpu.sync_copy(acc0, out.at[pl.ds(f0, N_PAD)])
    pltpu.sync_copy(acc1, out.at[pl.ds(f0 + N_PAD, N_PAD)])


@functools.cache
def _agg_call():
    return pl.kernel(
        _agg_body,
        out_type=jax.ShapeDtypeStruct((D_H * N_PAD,), jnp.float32),
        scratch_types=[
            pltpu.VMEM((CH,), jnp.int32),
            pltpu.VMEM((CH,), jnp.int32),
            pltpu.VMEM((N_PAD,), jnp.float32),
            pltpu.VMEM((N_PAD,), jnp.float32),
            pltpu.VMEM((N_PAD,), jnp.float32),
            pltpu.VMEM((N_PAD,), jnp.float32),
            pltpu.SemaphoreType.DMA,
            pltpu.SemaphoreType.DMA,
        ],
        **_sc_params(),
    )


def _prescale_kernel(xT_ref, w1t_ref, deg_ref, ys_ref, dinv_ref):
    i = pl.program_id(0)
    dsum = jnp.sum(deg_ref[...], axis=0, keepdims=True) + 1.0
    cols = lax.broadcasted_iota(jnp.int32, (1, BLKC), 1) + i * BLKC
    dinv = jnp.where(cols < N, lax.rsqrt(dsum), 0.0)
    dinv_ref[...] = dinv
    y = jnp.dot(w1t_ref[...], xT_ref[...], preferred_element_type=jnp.float32)
    ys_ref[...] = y * dinv


def _prescale(xT_pad, w1t, deg):
    return pl.pallas_call(
        _prescale_kernel,
        grid=(N_PAD // BLKC,),
        in_specs=[
            pl.BlockSpec((D_IN, BLKC), lambda i: (0, i)),
            pl.BlockSpec((D_H, D_IN), lambda i: (0, 0)),
            pl.BlockSpec((NW, BLKC), lambda i: (0, i)),
        ],
        out_specs=[
            pl.BlockSpec((D_H, BLKC), lambda i: (0, i)),
            pl.BlockSpec((1, BLKC), lambda i: (0, i)),
        ],
        out_shape=[
            jax.ShapeDtypeStruct((D_H, N_PAD), jnp.float32),
            jax.ShapeDtypeStruct((1, N_PAD), jnp.float32),
        ],
    )(xT_pad, w1t, deg)


def _mid_kernel(agg_ref, dinv_ref, b1_ref, w2t_ref, ys2_ref):
    dinv = dinv_ref[...]
    x1 = jnp.maximum(agg_ref[...] * dinv + b1_ref[...], 0.0)
    y2 = jnp.dot(w2t_ref[...], x1, preferred_element_type=jnp.float32)
    ys2_ref[...] = y2 * dinv


def _mid(aggT, dinv, b1col, w2t):
    return pl.pallas_call(
        _mid_kernel,
        grid=(N_PAD // BLKC,),
        in_specs=[
            pl.BlockSpec((D_H, BLKC), lambda i: (0, i)),
            pl.BlockSpec((1, BLKC), lambda i: (0, i)),
            pl.BlockSpec((D_H, 1), lambda i: (0, 0)),
            pl.BlockSpec((D_H, D_H), lambda i: (0, 0)),
        ],
        out_specs=pl.BlockSpec((D_H, BLKC), lambda i: (0, i)),
        out_shape=jax.ShapeDtypeStruct((D_H, N_PAD), jnp.float32),
    )(aggT, dinv, b1col, w2t)


def _head_kernel(agg_ref, dinv_ref, b2_ref, wpt_ref, bp_ref, out_ref):
    x2 = jnp.maximum(agg_ref[...] * dinv_ref[...] + b2_ref[...], 0.0)
    logits = jnp.dot(wpt_ref[...], x2, preferred_element_type=jnp.float32)
    logits = logits + bp_ref[...]
    rows = lax.broadcasted_iota(jnp.int32, (8, BLKC), 0)
    valid = rows < N_CLS
    m = jnp.max(jnp.where(valid, logits, -1e30), axis=0, keepdims=True)
    s = jnp.sum(jnp.where(valid, jnp.exp(logits - m), 0.0), axis=0, keepdims=True)
    out_ref[...] = logits - m - jnp.log(s)


def _head(aggT, dinv, b2col, wpt_pad, bp_col):
    return pl.pallas_call(
        _head_kernel,
        grid=(N_PAD // BLKC,),
        in_specs=[
            pl.BlockSpec((D_H, BLKC), lambda i: (0, i)),
            pl.BlockSpec((1, BLKC), lambda i: (0, i)),
            pl.BlockSpec((D_H, 1), lambda i: (0, 0)),
            pl.BlockSpec((8, D_H), lambda i: (0, 0)),
            pl.BlockSpec((8, 1), lambda i: (0, 0)),
        ],
        out_specs=pl.BlockSpec((8, BLKC), lambda i: (0, i)),
        out_shape=jax.ShapeDtypeStruct((8, N_PAD), jnp.float32),
    )(aggT, dinv, b2col, wpt_pad, bp_col)


def kernel(x, edge_index, W1, b1, W2, b2, Wp, bp):
    xT_pad = jnp.pad(x.T, ((0, 0), (0, N_PAD - N)))
    w1t = W1.T
    w2t = W2.T
    wpt_pad = jnp.pad(Wp.T, ((0, 8 - N_CLS), (0, 0)))
    bp_col = jnp.pad(bp, (0, 8 - N_CLS)).reshape(8, 1)
    b1col = b1.reshape(D_H, 1)
    b2col = b2.reshape(D_H, 1)

    pk1d = jnp.bitwise_or(edge_index[0],
                          jnp.left_shift(edge_index[1], 16))

    deg = _deg_call()(pk1d).reshape(NW, N_PAD)
    ys1T, dinv = _prescale(xT_pad, w1t, deg)
    agg1T = _agg_call()(ys1T.reshape(-1), pk1d).reshape(D_H, N_PAD)
    ys2T = _mid(agg1T, dinv, b1col, w2t)
    agg2T = _agg_call()(ys2T.reshape(-1), pk1d).reshape(D_H, N_PAD)
    outT = _head(agg2T, dinv, b2col, wpt_pad, bp_col)
    return outT[:N_CLS, :N].T
